# baseline (speedup 1.0000x reference)
import jax
import jax.numpy as jnp
from jax.experimental import pallas as pl


def _gate_body(c_ref, t_ref, wfc_ref, bfc_ref, wfc1_ref, bfc1_ref,
               mixed_ref, w_ref):
    cb = c_ref[...]
    tb = t_ref[...]
    wf = wfc_ref[...]
    h = (jnp.dot(cb, wf[:64, :], preferred_element_type=jnp.float32)
         + jnp.dot(tb, wf[64:, :], preferred_element_type=jnp.float32)
         + bfc_ref[...])
    h = jnp.where(h > 0, h, jnp.exp(jnp.minimum(h, 0.0)) - 1.0)
    h = jnp.dot(h, wfc1_ref[...], preferred_element_type=jnp.float32) \
        + bfc1_ref[...]
    w = jax.nn.sigmoid(h)
    w_ref[...] = w
    mixed_ref[...] = tb + (cb - tb) * w


def kernel(c, t, W_fc, b_fc, W_fc1, b_fc1):
    bs, n, dim = c.shape
    rows = bs * n
    c2 = c.reshape(rows, dim)
    t2 = t.reshape(rows, dim)
    bfc2 = b_fc.reshape(1, dim)
    bfc12 = b_fc1.reshape(1, dim)

    BN = 4000
    grid = (rows // BN,)

    row_spec = pl.BlockSpec((BN, dim), lambda i: (i, 0))
    rep = lambda shape: pl.BlockSpec(shape, lambda i: (0, 0))

    mixed, w = pl.pallas_call(
        _gate_body,
        grid=grid,
        in_specs=[
            row_spec,
            row_spec,
            rep((2 * dim, dim)),
            rep((1, dim)),
            rep((dim, dim)),
            rep((1, dim)),
        ],
        out_specs=[row_spec, row_spec],
        out_shape=[
            jax.ShapeDtypeStruct((rows, dim), jnp.float32),
            jax.ShapeDtypeStruct((rows, dim), jnp.float32),
        ],
    )(c2, t2, W_fc, bfc2, W_fc1, bfc12)

    return mixed.reshape(bs, n, dim), w.reshape(bs, n, dim)
